# baseline (speedup 1.0000x reference)
import jax
import jax.numpy as jnp
from jax import lax
from jax.experimental import pallas as pl
from jax.experimental.pallas import tpu as pltpu

B, T, D, E, C = 4096, 16, 1024, 8, 256
CH = 128
NBUF = 4
NCHUNK = B // CH


def _fused_body(sid_ref, x_hbm, w_ref, b_ref, out_ref, buf, sems):
    g = pl.program_id(0)

    @pl.when(g == 0)
    def _prime():
        for k in range(NBUF):
            pltpu.make_async_copy(
                x_hbm.at[pl.ds(k * CH, CH)], buf.at[k], sems.at[k]).start()

    slot = lax.rem(g, NBUF)
    pltpu.make_async_copy(
        x_hbm.at[pl.ds(g * CH, CH)], buf.at[slot], sems.at[slot]).wait()

    xp = jnp.sum(buf[slot], axis=1) * (1.0 / T)
    acc = lax.dot_general(
        xp, w_ref[...],
        dimension_numbers=(((1,), (1,)), ((), ())),
        preferred_element_type=jnp.float32,
    )
    sid = sid_ref[0, 0, :]
    out = jnp.zeros((CH, C), dtype=jnp.float32)
    for e in range(E):
        mask = (sid == e).astype(jnp.float32)[:, None]
        out = out + mask * (acc[:, e * C:(e + 1) * C] + b_ref[e, :][None, :])
    out_ref[...] = out

    @pl.when(g + NBUF < NCHUNK)
    def _refill():
        pltpu.make_async_copy(
            x_hbm.at[pl.ds((g + NBUF) * CH, CH)],
            buf.at[slot], sems.at[slot]).start()


def kernel(x, system_id, W, b):
    sid3 = system_id.astype(jnp.int32).reshape(NCHUNK, 1, CH)
    wcat = W.reshape(E * C, D)
    return pl.pallas_call(
        _fused_body,
        grid=(NCHUNK,),
        in_specs=[
            pl.BlockSpec((1, 1, CH), lambda g: (g, 0, 0)),
            pl.BlockSpec(memory_space=pltpu.MemorySpace.HBM),
            pl.BlockSpec((E * C, D), lambda g: (0, 0)),
            pl.BlockSpec((E, C), lambda g: (0, 0)),
        ],
        out_specs=pl.BlockSpec((CH, C), lambda g: (g, 0)),
        out_shape=jax.ShapeDtypeStruct((B, C), jnp.float32),
        scratch_shapes=[
            pltpu.VMEM((NBUF, CH, T, D), jnp.float32),
            pltpu.SemaphoreType.DMA((NBUF,)),
        ],
        compiler_params=pltpu.CompilerParams(
            dimension_semantics=("arbitrary",),
        ),
    )(sid3, x, wcat, b)


# speedup vs baseline: 1.0410x; 1.0410x over previous
import jax
import jax.numpy as jnp
from jax import lax
from jax.experimental import pallas as pl
from jax.experimental.pallas import tpu as pltpu

B, T, D, E, C = 4096, 16, 1024, 8, 256
CH = 256
NBUF = 3
NCHUNK = B // CH


def _fused_body(sid_ref, x_hbm, w_ref, b_ref, out_ref, buf, sems):
    g = pl.program_id(0)

    @pl.when(g == 0)
    def _prime():
        for k in range(NBUF):
            pltpu.make_async_copy(
                x_hbm.at[pl.ds(k * CH, CH)], buf.at[k], sems.at[k]).start()

    slot = lax.rem(g, NBUF)
    pltpu.make_async_copy(
        x_hbm.at[pl.ds(g * CH, CH)], buf.at[slot], sems.at[slot]).wait()

    xp = jnp.sum(buf[slot], axis=1) * (1.0 / T)
    acc = lax.dot_general(
        xp, w_ref[...],
        dimension_numbers=(((1,), (1,)), ((), ())),
        preferred_element_type=jnp.float32,
    )
    sid = sid_ref[0, 0, :]
    out = jnp.zeros((CH, C), dtype=jnp.float32)
    for e in range(E):
        mask = (sid == e).astype(jnp.float32)[:, None]
        out = out + mask * (acc[:, e * C:(e + 1) * C] + b_ref[e, :][None, :])
    out_ref[...] = out

    @pl.when(g + NBUF < NCHUNK)
    def _refill():
        pltpu.make_async_copy(
            x_hbm.at[pl.ds((g + NBUF) * CH, CH)],
            buf.at[slot], sems.at[slot]).start()


def kernel(x, system_id, W, b):
    sid3 = system_id.astype(jnp.int32).reshape(NCHUNK, 1, CH)
    wcat = W.reshape(E * C, D)
    return pl.pallas_call(
        _fused_body,
        grid=(NCHUNK,),
        in_specs=[
            pl.BlockSpec((1, 1, CH), lambda g: (g, 0, 0)),
            pl.BlockSpec(memory_space=pltpu.MemorySpace.HBM),
            pl.BlockSpec((E * C, D), lambda g: (0, 0)),
            pl.BlockSpec((E, C), lambda g: (0, 0)),
        ],
        out_specs=pl.BlockSpec((CH, C), lambda g: (g, 0)),
        out_shape=jax.ShapeDtypeStruct((B, C), jnp.float32),
        scratch_shapes=[
            pltpu.VMEM((NBUF, CH, T, D), jnp.float32),
            pltpu.SemaphoreType.DMA((NBUF,)),
        ],
        compiler_params=pltpu.CompilerParams(
            dimension_semantics=("arbitrary",),
        ),
    )(sid3, x, wcat, b)


# speedup vs baseline: 1.1247x; 1.0804x over previous
import jax
import jax.numpy as jnp
from jax import lax
from jax.experimental import pallas as pl
from jax.experimental.pallas import tpu as pltpu

B, T, D, E, C = 4096, 16, 1024, 8, 256
BLK = 256


def _fused_body(sid_ref, x_ref, w_ref, b_ref, out_ref):
    xp = jnp.sum(x_ref[...], axis=1) * (1.0 / T)
    acc = lax.dot_general(
        xp, w_ref[...],
        dimension_numbers=(((1,), (1,)), ((), ())),
        preferred_element_type=jnp.float32,
    )
    sid = sid_ref[0, 0, :]
    out = jnp.zeros((BLK, C), dtype=jnp.float32)
    for e in range(E):
        mask = (sid == e).astype(jnp.float32)[:, None]
        out = out + mask * (acc[:, e * C:(e + 1) * C] + b_ref[e, :][None, :])
    out_ref[...] = out


def kernel(x, system_id, W, b):
    nblk = B // BLK
    sid3 = system_id.astype(jnp.int32).reshape(nblk, 1, BLK)
    wcat = W.reshape(E * C, D)
    return pl.pallas_call(
        _fused_body,
        grid=(nblk,),
        in_specs=[
            pl.BlockSpec((1, 1, BLK), lambda g: (g, 0, 0)),
            pl.BlockSpec((BLK, T, D), lambda g: (g, 0, 0)),
            pl.BlockSpec((E * C, D), lambda g: (0, 0)),
            pl.BlockSpec((E, C), lambda g: (0, 0)),
        ],
        out_specs=pl.BlockSpec((BLK, C), lambda g: (g, 0)),
        out_shape=jax.ShapeDtypeStruct((B, C), jnp.float32),
        compiler_params=pltpu.CompilerParams(
            dimension_semantics=("arbitrary",),
        ),
    )(sid3, x, wcat, b)
